# Initial kernel scaffold; baseline (speedup 1.0000x reference)
import jax
import jax.numpy as jnp
from jax.experimental import pallas as pl


def kernel(flat_origin_curves, flat_new_curves, dict_points, dict_ref, dict_bool):
    raise NotImplementedError("write your pallas kernel here")


# speedup vs baseline: 1.6966x; 1.6966x over previous
import functools

import jax
import jax.numpy as jnp
from jax.experimental import pallas as pl
from jax.experimental.pallas import tpu as pltpu

N = 4096
K = 8192
BN = 256
BK = 2048
NB = N // BN
KB = K // BK


def _body(q_ref, nw_ref, rxy_ref, pxy_ref, brow_ref, out_ref,
          minv, selx, sely, selb, acc):
    nb = pl.program_id(0)
    kb = pl.program_id(1)

    @pl.when(kb == 0)
    def _init():
        minv[...] = jnp.full((BN, 1), jnp.inf, jnp.float32)
        selx[...] = jnp.zeros((BN, 1), jnp.float32)
        sely[...] = jnp.zeros((BN, 1), jnp.float32)
        selb[...] = jnp.zeros((BN, 1), jnp.float32)

    qx = q_ref[:, 0:1]
    qy = q_ref[:, 1:2]
    rx = rxy_ref[0:1, :]
    ry = rxy_ref[1:2, :]
    dx = rx - qx
    dy = ry - qy
    d2 = dx * dx + dy * dy

    m = jnp.min(d2, axis=1, keepdims=True)
    iota = jax.lax.broadcasted_iota(jnp.int32, (BN, BK), 1)
    lidx = jnp.min(jnp.where(d2 <= m, iota, BK), axis=1, keepdims=True)
    eq = iota == lidx
    cpx = jnp.sum(jnp.where(eq, pxy_ref[0:1, :], 0.0), axis=1, keepdims=True)
    cpy = jnp.sum(jnp.where(eq, pxy_ref[1:2, :], 0.0), axis=1, keepdims=True)
    cb = jnp.sum(jnp.where(eq, brow_ref[0:1, :], 0.0), axis=1, keepdims=True)

    upd = m < minv[...]
    selx[...] = jnp.where(upd, cpx, selx[...])
    sely[...] = jnp.where(upd, cpy, sely[...])
    selb[...] = jnp.where(upd, cb, selb[...])
    minv[...] = jnp.where(upd, m, minv[...])

    @pl.when(kb == KB - 1)
    def _finish():
        dxn = nw_ref[:, 0:1] - selx[...]
        dyn = nw_ref[:, 1:2] - sely[...]
        pp = jnp.sqrt(dxn * dxn + dyn * dyn)
        b = selb[...]
        s = jnp.sum(pp * b)
        c = jnp.sum(b)

        @pl.when(nb == 0)
        def _zero():
            acc[0] = 0.0
            acc[1] = 0.0

        acc[0] = acc[0] + s
        acc[1] = acc[1] + c

        @pl.when(nb == NB - 1)
        def _emit():
            out_ref[0, 0] = acc[0] / acc[1]


@functools.partial(jax.jit, static_argnames=())
def _track_loss(flat_origin_curves, flat_new_curves, rxy, pxy, brow):
    out = pl.pallas_call(
        _body,
        grid=(NB, KB),
        in_specs=[
            pl.BlockSpec((BN, 2), lambda nb, kb: (nb, 0)),
            pl.BlockSpec((BN, 2), lambda nb, kb: (nb, 0)),
            pl.BlockSpec((2, BK), lambda nb, kb: (0, kb)),
            pl.BlockSpec((2, BK), lambda nb, kb: (0, kb)),
            pl.BlockSpec((1, BK), lambda nb, kb: (0, kb)),
        ],
        out_specs=pl.BlockSpec(memory_space=pltpu.SMEM),
        out_shape=jax.ShapeDtypeStruct((1, 1), jnp.float32),
        scratch_shapes=[
            pltpu.VMEM((BN, 1), jnp.float32),
            pltpu.VMEM((BN, 1), jnp.float32),
            pltpu.VMEM((BN, 1), jnp.float32),
            pltpu.VMEM((BN, 1), jnp.float32),
            pltpu.SMEM((2,), jnp.float32),
        ],
        compiler_params=pltpu.CompilerParams(
            dimension_semantics=("arbitrary", "arbitrary"),
        ),
    )(flat_origin_curves, flat_new_curves, rxy, pxy, brow)
    return out[0, 0]


def kernel(flat_origin_curves, flat_new_curves, dict_points, dict_ref, dict_bool):
    rxy = dict_ref.T
    pxy = dict_points.T
    brow = dict_bool.astype(jnp.float32)[None, :]
    return _track_loss(flat_origin_curves, flat_new_curves, rxy, pxy, brow)


# speedup vs baseline: 2.0769x; 1.2242x over previous
import functools

import jax
import jax.numpy as jnp
from jax import lax
from jax.experimental import pallas as pl
from jax.experimental.pallas import tpu as pltpu
from jax.experimental.pallas import tpu_sc as plsc

N = 4096
K = 8192
BN = 256
BK = 2048
NB = N // BN
KB = K // BK

NC = 2
NS = 16
NW = NC * NS
QPW = N // NW
L = 16


def _argmin_body(q_ref, rxy_ref, idx_ref, minv, mini):
    kb = pl.program_id(1)

    @pl.when(kb == 0)
    def _init():
        minv[...] = jnp.full((BN, 1), jnp.inf, jnp.float32)
        mini[...] = jnp.zeros((BN, 1), jnp.int32)

    qx = q_ref[:, 0:1]
    qy = q_ref[:, 1:2]
    rx = rxy_ref[0:1, :]
    ry = rxy_ref[1:2, :]
    dx = rx - qx
    dy = ry - qy
    d2 = dx * dx + dy * dy

    m = jnp.min(d2, axis=1, keepdims=True)
    iota = jax.lax.broadcasted_iota(jnp.int32, (BN, BK), 1)
    lidx = jnp.min(jnp.where(d2 <= m, iota, BK), axis=1, keepdims=True)

    upd = m < minv[...]
    mini[...] = jnp.where(upd, lidx + kb * BK, mini[...])
    minv[...] = jnp.where(upd, m, minv[...])

    @pl.when(kb == KB - 1)
    def _emit():
        idx_ref[...] = mini[...]


def _sc_gather_body(idx_hbm, px_hbm, py_hbm, b_hbm, nx_hbm, ny_hbm,
                    d2_out, b_out,
                    idx_v, px_v, py_v, b_v, nx_v, ny_v, d2_v, gb_v):
    wid = lax.axis_index("s") * NC + lax.axis_index("c")
    base = wid * QPW
    pltpu.sync_copy(idx_hbm.at[pl.ds(base, QPW)], idx_v)
    pltpu.sync_copy(px_hbm, px_v)
    pltpu.sync_copy(py_hbm, py_v)
    pltpu.sync_copy(b_hbm, b_v)
    pltpu.sync_copy(nx_hbm.at[pl.ds(base, QPW)], nx_v)
    pltpu.sync_copy(ny_hbm.at[pl.ds(base, QPW)], ny_v)
    for j in range(QPW // L):
        sl = pl.ds(j * L, L)
        iv = idx_v[sl]
        gx = plsc.load_gather(px_v, [iv])
        gy = plsc.load_gather(py_v, [iv])
        gb = plsc.load_gather(b_v, [iv])
        dx = nx_v[sl] - gx
        dy = ny_v[sl] - gy
        d2_v[sl] = dx * dx + dy * dy
        gb_v[sl] = gb
    pltpu.sync_copy(d2_v, d2_out.at[wid])
    pltpu.sync_copy(gb_v, b_out.at[wid])


def _reduce_body(d2_ref, b_ref, out_ref):
    pp = jnp.sqrt(d2_ref[...])
    b = b_ref[...]
    out_ref[0, 0] = jnp.sum(pp * b) / jnp.sum(b)


@jax.jit
def _track_loss(q, rxy, px, py, bf, nx, ny):
    idx2d = pl.pallas_call(
        _argmin_body,
        grid=(NB, KB),
        in_specs=[
            pl.BlockSpec((BN, 2), lambda nb, kb: (nb, 0)),
            pl.BlockSpec((2, BK), lambda nb, kb: (0, kb)),
        ],
        out_specs=pl.BlockSpec((BN, 1), lambda nb, kb: (nb, 0)),
        out_shape=jax.ShapeDtypeStruct((N, 1), jnp.int32),
        scratch_shapes=[
            pltpu.VMEM((BN, 1), jnp.float32),
            pltpu.VMEM((BN, 1), jnp.int32),
        ],
        compiler_params=pltpu.CompilerParams(
            dimension_semantics=("arbitrary", "arbitrary"),
        ),
    )(q, rxy)
    idx = idx2d.reshape(N)

    sc_gather = functools.partial(
        pl.kernel,
        out_type=(
            jax.ShapeDtypeStruct((NW, QPW), jnp.float32),
            jax.ShapeDtypeStruct((NW, QPW), jnp.float32),
        ),
        mesh=plsc.VectorSubcoreMesh(core_axis_name="c", subcore_axis_name="s"),
        compiler_params=pltpu.CompilerParams(needs_layout_passes=False),
        scratch_types=[
            pltpu.VMEM((QPW,), jnp.int32),
            pltpu.VMEM((K,), jnp.float32),
            pltpu.VMEM((K,), jnp.float32),
            pltpu.VMEM((K,), jnp.float32),
            pltpu.VMEM((QPW,), jnp.float32),
            pltpu.VMEM((QPW,), jnp.float32),
            pltpu.VMEM((QPW,), jnp.float32),
            pltpu.VMEM((QPW,), jnp.float32),
        ],
    )(_sc_gather_body)
    d2g, bg = sc_gather(idx, px, py, bf, nx, ny)

    out = pl.pallas_call(
        _reduce_body,
        out_specs=pl.BlockSpec(memory_space=pltpu.SMEM),
        out_shape=jax.ShapeDtypeStruct((1, 1), jnp.float32),
    )(d2g, bg)
    return out[0, 0]


def kernel(flat_origin_curves, flat_new_curves, dict_points, dict_ref, dict_bool):
    rxy = dict_ref.T
    px = dict_points[:, 0]
    py = dict_points[:, 1]
    bf = dict_bool.astype(jnp.float32)
    nx = flat_new_curves[:, 0]
    ny = flat_new_curves[:, 1]
    return _track_loss(flat_origin_curves, rxy, px, py, bf, nx, ny)


# speedup vs baseline: 2.5172x; 1.2120x over previous
import functools

import jax
import jax.numpy as jnp
from jax import lax
from jax.experimental import pallas as pl
from jax.experimental.pallas import tpu as pltpu
from jax.experimental.pallas import tpu_sc as plsc

N = 4096
K = 8192
BN = 256
BK = 2048
NB = N // BN
KB = K // BK

NC = 2
NS = 16
NW = NC * NS
QPW = N // NW
L = 16


def _argmin_body(q_ref, rxy_ref, idx_ref):
    qx = q_ref[:, 0:1]
    qy = q_ref[:, 1:2]
    rx = rxy_ref[0:1, :]
    ry = rxy_ref[1:2, :]
    dx = rx - qx
    dy = ry - qy
    d2 = dx * dx + dy * dy

    m = jnp.min(d2, axis=1, keepdims=True)
    fiota = jax.lax.broadcasted_iota(jnp.int32, (BN, K), 1).astype(jnp.float32)
    fidx = jnp.min(jnp.where(d2 <= m, fiota, float(K)), axis=1, keepdims=True)
    idx_ref[...] = fidx.astype(jnp.int32)


def _sc_gather_body(idx_hbm, px_hbm, py_hbm, b_hbm, nx_hbm, ny_hbm,
                    d2_out, b_out,
                    idx_v, px_v, py_v, b_v, nx_v, ny_v, d2_v, gb_v):
    wid = lax.axis_index("s") * NC + lax.axis_index("c")
    base = wid * QPW
    pltpu.sync_copy(idx_hbm.at[pl.ds(base, QPW)], idx_v)
    pltpu.sync_copy(px_hbm, px_v)
    pltpu.sync_copy(py_hbm, py_v)
    pltpu.sync_copy(b_hbm, b_v)
    pltpu.sync_copy(nx_hbm.at[pl.ds(base, QPW)], nx_v)
    pltpu.sync_copy(ny_hbm.at[pl.ds(base, QPW)], ny_v)
    for j in range(QPW // L):
        sl = pl.ds(j * L, L)
        iv = idx_v[sl]
        gx = plsc.load_gather(px_v, [iv])
        gy = plsc.load_gather(py_v, [iv])
        gb = plsc.load_gather(b_v, [iv])
        dx = nx_v[sl] - gx
        dy = ny_v[sl] - gy
        d2_v[sl] = dx * dx + dy * dy
        gb_v[sl] = gb
    pltpu.sync_copy(d2_v, d2_out.at[wid])
    pltpu.sync_copy(gb_v, b_out.at[wid])


def _reduce_body(d2_ref, b_ref, out_ref):
    pp = jnp.sqrt(d2_ref[...])
    b = b_ref[...]
    out_ref[0, 0] = jnp.sum(pp * b) / jnp.sum(b)


@jax.jit
def _track_loss(q, rxy, px, py, bf, nx, ny):
    idx2d = pl.pallas_call(
        _argmin_body,
        grid=(NB,),
        in_specs=[
            pl.BlockSpec((BN, 2), lambda nb: (nb, 0)),
            pl.BlockSpec((2, K), lambda nb: (0, 0)),
        ],
        out_specs=pl.BlockSpec((BN, 1), lambda nb: (nb, 0)),
        out_shape=jax.ShapeDtypeStruct((N, 1), jnp.int32),
        compiler_params=pltpu.CompilerParams(
            dimension_semantics=("arbitrary",),
        ),
    )(q, rxy)
    idx = idx2d.reshape(N)

    sc_gather = functools.partial(
        pl.kernel,
        out_type=(
            jax.ShapeDtypeStruct((NW, QPW), jnp.float32),
            jax.ShapeDtypeStruct((NW, QPW), jnp.float32),
        ),
        mesh=plsc.VectorSubcoreMesh(core_axis_name="c", subcore_axis_name="s"),
        compiler_params=pltpu.CompilerParams(needs_layout_passes=False),
        scratch_types=[
            pltpu.VMEM((QPW,), jnp.int32),
            pltpu.VMEM((K,), jnp.float32),
            pltpu.VMEM((K,), jnp.float32),
            pltpu.VMEM((K,), jnp.float32),
            pltpu.VMEM((QPW,), jnp.float32),
            pltpu.VMEM((QPW,), jnp.float32),
            pltpu.VMEM((QPW,), jnp.float32),
            pltpu.VMEM((QPW,), jnp.float32),
        ],
    )(_sc_gather_body)
    d2g, bg = sc_gather(idx, px, py, bf, nx, ny)

    out = pl.pallas_call(
        _reduce_body,
        out_specs=pl.BlockSpec(memory_space=pltpu.SMEM),
        out_shape=jax.ShapeDtypeStruct((1, 1), jnp.float32),
    )(d2g, bg)
    return out[0, 0]


def kernel(flat_origin_curves, flat_new_curves, dict_points, dict_ref, dict_bool):
    rxy = dict_ref.T
    px = dict_points[:, 0]
    py = dict_points[:, 1]
    bf = dict_bool.astype(jnp.float32)
    nx = flat_new_curves[:, 0]
    ny = flat_new_curves[:, 1]
    return _track_loss(flat_origin_curves, rxy, px, py, bf, nx, ny)


# speedup vs baseline: 2.6817x; 1.0654x over previous
import functools

import jax
import jax.numpy as jnp
from jax import lax
from jax.experimental import pallas as pl
from jax.experimental.pallas import tpu as pltpu
from jax.experimental.pallas import tpu_sc as plsc

N = 4096
K = 8192
BN = 256
NB = N // BN

NC = 2
NS = 16
NW = NC * NS
QPW = N // NW
L = 16


def _argmin_body(q_ref, rt_ref, idx_ref, rhs8):
    nb = pl.program_id(0)

    @pl.when(nb == 0)
    def _build_rhs():
        rx = rt_ref[0:1, :]
        ry = rt_ref[1:2, :]
        rhs8[0:2, :] = rt_ref[...]
        rhs8[2:3, :] = rx * rx + ry * ry
        rhs8[3:8, :] = jnp.zeros((5, K), jnp.float32)

    lhs8 = jnp.concatenate(
        [q_ref[...] * -2.0, jnp.ones((BN, 1), jnp.float32),
         jnp.zeros((BN, 5), jnp.float32)], axis=1)
    score = jax.lax.dot_general(
        lhs8, rhs8[...], (((1,), (0,)), ((), ())),
        preferred_element_type=jnp.float32)

    m = jnp.min(score, axis=1, keepdims=True)
    fiota = jax.lax.broadcasted_iota(jnp.int32, (BN, K), 1).astype(jnp.float32)
    fidx = jnp.min(jnp.where(score <= m, fiota, float(K)), axis=1,
                   keepdims=True)
    idx_ref[...] = fidx.astype(jnp.int32)


def _sc_gather_body(idx_hbm, dp_hbm, b_hbm, nw_hbm,
                    d2_out, b_out,
                    idx_v, dp_v, b_v, nw_v, d2_v, gb_v):
    wid = lax.axis_index("s") * NC + lax.axis_index("c")
    base = wid * QPW
    pltpu.sync_copy(idx_hbm.at[pl.ds(base, QPW)], idx_v)
    pltpu.sync_copy(dp_hbm, dp_v)
    pltpu.sync_copy(b_hbm, b_v)
    pltpu.sync_copy(nw_hbm.at[pl.ds(2 * base, 2 * QPW)], nw_v)
    li = lax.iota(jnp.int32, L)
    for j in range(QPW // L):
        sl = pl.ds(j * L, L)
        iv = idx_v[sl]
        iv2 = iv * 2
        gx = plsc.load_gather(dp_v, [iv2])
        gy = plsc.load_gather(dp_v, [iv2 + 1])
        gb = plsc.load_gather(b_v, [iv])
        nl = li * 2 + (2 * L) * j
        nx = plsc.load_gather(nw_v, [nl])
        ny = plsc.load_gather(nw_v, [nl + 1])
        dx = nx - gx
        dy = ny - gy
        d2_v[sl] = dx * dx + dy * dy
        gb_v[sl] = gb
    pltpu.sync_copy(d2_v, d2_out.at[wid])
    pltpu.sync_copy(gb_v, b_out.at[wid])


def _reduce_body(d2_ref, b_ref, out_ref):
    pp = jnp.sqrt(d2_ref[...])
    b = b_ref[...]
    out_ref[0, 0] = jnp.sum(pp * b) / jnp.sum(b)


@jax.jit
def _track_loss(q, rt, dpflat, bf, nwflat):
    idx2d = pl.pallas_call(
        _argmin_body,
        grid=(NB,),
        in_specs=[
            pl.BlockSpec((BN, 2), lambda nb: (nb, 0)),
            pl.BlockSpec((2, K), lambda nb: (0, 0)),
        ],
        out_specs=pl.BlockSpec((BN, 1), lambda nb: (nb, 0)),
        out_shape=jax.ShapeDtypeStruct((N, 1), jnp.int32),
        scratch_shapes=[
            pltpu.VMEM((8, K), jnp.float32),
        ],
        compiler_params=pltpu.CompilerParams(
            dimension_semantics=("arbitrary",),
        ),
    )(q, rt)
    idx = idx2d.reshape(N)

    sc_gather = functools.partial(
        pl.kernel,
        out_type=(
            jax.ShapeDtypeStruct((NW, QPW), jnp.float32),
            jax.ShapeDtypeStruct((NW, QPW), jnp.float32),
        ),
        mesh=plsc.VectorSubcoreMesh(core_axis_name="c", subcore_axis_name="s"),
        compiler_params=pltpu.CompilerParams(needs_layout_passes=False),
        scratch_types=[
            pltpu.VMEM((QPW,), jnp.int32),
            pltpu.VMEM((2 * K,), jnp.float32),
            pltpu.VMEM((K,), jnp.float32),
            pltpu.VMEM((2 * QPW,), jnp.float32),
            pltpu.VMEM((QPW,), jnp.float32),
            pltpu.VMEM((QPW,), jnp.float32),
        ],
    )(_sc_gather_body)
    d2g, bg = sc_gather(idx, dpflat, bf, nwflat)

    out = pl.pallas_call(
        _reduce_body,
        out_specs=pl.BlockSpec(memory_space=pltpu.SMEM),
        out_shape=jax.ShapeDtypeStruct((1, 1), jnp.float32),
    )(d2g, bg)
    return out[0, 0]


def kernel(flat_origin_curves, flat_new_curves, dict_points, dict_ref, dict_bool):
    dpflat = dict_points.reshape(2 * K)
    nwflat = flat_new_curves.reshape(2 * N)
    bf = dict_bool.astype(jnp.float32)
    return _track_loss(flat_origin_curves, dict_ref.T, dpflat, bf, nwflat)
